# baseline (device time: 86776 ns/iter reference)
import jax
import jax.numpy as jnp
from jax import lax
from jax.experimental import pallas as pl
from jax.experimental.pallas import tpu as pltpu

N_DEV = 16
N_STEP = 8


def kernel(x, w_mat, scale_x, scale_w):
    m_per, k = x.shape
    _, n = w_mat.shape
    n_per = n // N_DEV
    n_blk = 2 * n_per
    m_full = m_per * N_DEV

    def body(x_ref, w_hbm, sx_ref, sw_ref, out_ref,
             xbf, wbuf, sstage, rstage, wsems, send_sem, recv_sem):
        my = lax.axis_index("i")
        s = sx_ref[0] * sw_ref[0]

        def wcopy(t, slot):
            b = lax.rem(my // 2 + t, N_STEP)
            return pltpu.make_async_copy(
                w_hbm.at[:, pl.ds(b * n_blk, n_blk)],
                wbuf.at[slot],
                wsems.at[slot],
            )

        wcopy(0, 0).start()
        xbf[...] = x_ref[...].astype(jnp.bfloat16)

        def send_to(tgt):
            return pltpu.make_async_remote_copy(
                src_ref=sstage.at[tgt],
                dst_ref=rstage.at[my],
                send_sem=send_sem,
                recv_sem=recv_sem,
                device_id=(tgt,),
                device_id_type=pl.DeviceIdType.MESH,
            )

        for t in range(N_STEP):
            slot = t % 2
            if t + 1 < N_STEP:
                wcopy(t + 1, (t + 1) % 2).start()
            wcopy(t, slot).wait()
            acc = jnp.dot(
                xbf[...],
                wbuf[slot].astype(jnp.bfloat16),
                preferred_element_type=jnp.float32,
            )
            y = jnp.maximum(acc * s, 0.0)
            tgt0 = 2 * lax.rem(my // 2 + t, N_STEP)
            sstage[tgt0] = y[:, :n_per].astype(jnp.bfloat16)
            sstage[tgt0 + 1] = y[:, n_per:].astype(jnp.bfloat16)
            if t == 0:
                @pl.when(lax.rem(my, 2) == 0)
                def _():
                    out_ref[pl.ds(my * m_per, m_per)] = y[:, :n_per]

                @pl.when(lax.rem(my, 2) == 1)
                def _():
                    out_ref[pl.ds(my * m_per, m_per)] = y[:, n_per:]

                send_to(jnp.bitwise_xor(my, 1)).start()
            else:
                send_to(tgt0).start()
                send_to(tgt0 + 1).start()

        done = pltpu.make_async_remote_copy(
            src_ref=sstage.at[0],
            dst_ref=rstage.at[0],
            send_sem=send_sem,
            recv_sem=recv_sem,
            device_id=(my,),
            device_id_type=pl.DeviceIdType.MESH,
        )
        for _ in range(N_DEV - 1):
            done.wait_recv()
        for p in range(N_DEV):
            @pl.when(p != my)
            def _(p=p):
                out_ref[pl.ds(p * m_per, m_per)] = rstage[p].astype(jnp.float32)
        for _ in range(N_DEV - 1):
            done.wait_send()

    out_shape = jax.ShapeDtypeStruct((m_full, n_per), jnp.float32)
    return pl.pallas_call(
        body,
        out_shape=out_shape,
        in_specs=[
            pl.BlockSpec(memory_space=pltpu.MemorySpace.VMEM),
            pl.BlockSpec(memory_space=pltpu.MemorySpace.HBM),
            pl.BlockSpec(memory_space=pltpu.MemorySpace.SMEM),
            pl.BlockSpec(memory_space=pltpu.MemorySpace.SMEM),
        ],
        out_specs=pl.BlockSpec(memory_space=pltpu.MemorySpace.VMEM),
        scratch_shapes=[
            pltpu.VMEM((m_per, k), jnp.bfloat16),
            pltpu.VMEM((2, k, n_blk), jnp.float32),
            pltpu.VMEM((N_DEV, m_per, n_per), jnp.bfloat16),
            pltpu.VMEM((N_DEV, m_per, n_per), jnp.bfloat16),
            pltpu.SemaphoreType.DMA((2,)),
            pltpu.SemaphoreType.DMA,
            pltpu.SemaphoreType.DMA,
        ],
        compiler_params=pltpu.CompilerParams(
            vmem_limit_bytes=60 * 1024 * 1024,
        ),
    )(x, w_mat, scale_x, scale_w)


# device time: 59977 ns/iter; 1.4468x vs baseline; 1.4468x over previous
import jax
import jax.numpy as jnp
from jax import lax
from jax.experimental import pallas as pl
from jax.experimental.pallas import tpu as pltpu

N_DEV = 16
N_SLOT = 4


def kernel(x, w_mat, scale_x, scale_w):
    m_per, k = x.shape
    _, n = w_mat.shape
    n_per = n // N_DEV
    m_full = m_per * N_DEV

    def body(x_ref, w_hbm, sx_ref, sw_ref, out_ref,
             xbf, wbuf, sstage, wsems, send_sem, recv_sem):
        my = lax.axis_index("i")
        s = sx_ref[0] * sw_ref[0]

        zm = my // 4
        q = lax.rem(my, 4)
        lo = zm < 2
        plane_order = [
            jnp.where(lo, 3, 0),
            jnp.where(lo, 2, 1),
            jnp.bitwise_xor(zm, 1),
            zm,
        ]

        def tgt_of(j):
            c, r = divmod(j - 1, 4)
            if c < 3:
                return plane_order[c] * 4 + lax.rem(q + r, 4)
            return zm * 4 + lax.rem(q + r + 1, 4)

        def blk_of(j):
            return tgt_of(j) if j < N_DEV else my

        def wcopy(j):
            return pltpu.make_async_copy(
                w_hbm.at[:, pl.ds(blk_of(j) * n_per, n_per)],
                wbuf.at[(j - 1) % N_SLOT],
                wsems.at[(j - 1) % N_SLOT],
            )

        for j in range(1, N_SLOT):
            wcopy(j).start()
        xbf[...] = x_ref[...].astype(jnp.bfloat16)

        barrier_sem = pltpu.get_barrier_semaphore()
        for off in range(1, N_DEV):
            pl.semaphore_signal(
                barrier_sem, inc=1,
                device_id=(lax.rem(my + off, N_DEV),),
                device_id_type=pl.DeviceIdType.MESH,
            )
        pl.semaphore_wait(barrier_sem, N_DEV - 1)

        for j in range(1, N_DEV + 1):
            if j + N_SLOT - 1 <= N_DEV:
                wcopy(j + N_SLOT - 1).start()
            wcopy(j).wait()
            acc = jnp.dot(
                xbf[...],
                wbuf[(j - 1) % N_SLOT].astype(jnp.bfloat16),
                preferred_element_type=jnp.float32,
            )
            chunk = jnp.maximum(acc * s, 0.0).astype(jnp.bfloat16)
            if j < N_DEV:
                tgt = tgt_of(j)
                sstage[tgt] = chunk
                rdma = pltpu.make_async_remote_copy(
                    src_ref=sstage.at[tgt],
                    dst_ref=out_ref.at[pl.ds(my * m_per, m_per)],
                    send_sem=send_sem,
                    recv_sem=recv_sem,
                    device_id=(tgt,),
                    device_id_type=pl.DeviceIdType.MESH,
                )
                rdma.start()
            else:
                out_ref[pl.ds(my * m_per, m_per)] = chunk

        done = pltpu.make_async_remote_copy(
            src_ref=sstage.at[0],
            dst_ref=out_ref.at[pl.ds(0, m_per)],
            send_sem=send_sem,
            recv_sem=recv_sem,
            device_id=(my,),
            device_id_type=pl.DeviceIdType.MESH,
        )
        for _ in range(N_DEV - 1):
            done.wait_recv()
        for _ in range(N_DEV - 1):
            done.wait_send()

    out_shape = jax.ShapeDtypeStruct((m_full, n_per), jnp.bfloat16)
    return pl.pallas_call(
        body,
        out_shape=out_shape,
        in_specs=[
            pl.BlockSpec(memory_space=pltpu.MemorySpace.VMEM),
            pl.BlockSpec(memory_space=pltpu.MemorySpace.HBM),
            pl.BlockSpec(memory_space=pltpu.MemorySpace.SMEM),
            pl.BlockSpec(memory_space=pltpu.MemorySpace.SMEM),
        ],
        out_specs=pl.BlockSpec(memory_space=pltpu.MemorySpace.VMEM),
        scratch_shapes=[
            pltpu.VMEM((m_per, k), jnp.bfloat16),
            pltpu.VMEM((N_SLOT, k, n_per), jnp.float32),
            pltpu.VMEM((N_DEV, m_per, n_per), jnp.bfloat16),
            pltpu.SemaphoreType.DMA((N_SLOT,)),
            pltpu.SemaphoreType.DMA,
            pltpu.SemaphoreType.DMA,
        ],
        compiler_params=pltpu.CompilerParams(
            vmem_limit_bytes=48 * 1024 * 1024,
            collective_id=0,
        ),
    )(x, w_mat, scale_x, scale_w)
